# baseline (speedup 1.0000x reference)
import jax
import jax.numpy as jnp
from jax import lax
from jax.experimental import pallas as pl
from jax.experimental.pallas import tpu as pltpu
from jax.experimental.pallas import tpu_sc as plsc

N = 10000
D = 128
E = 320000
CELL_LEN = 16
SAME_W = 0.3
CROSS_W = 1.0

NC = 2
NS = 16
NW = NC * NS
L = 16

EPT = E // NW
C = 80
NCH = EPT // C
N_PAD = 10240
RPT = N_PAD // NS

C2 = 80
NCH2 = 125
IDX_BITS = 14

_MESH = dict(core_axis_name="c", subcore_axis_name="s",
             num_cores=NC, num_subcores=NS)


def _zero_fill(buf, nrows):
    zero16 = jnp.zeros((L,), jnp.float32)

    def fill(r, _):
        for f in range(D // L):
            buf[r, pl.ds(f * L, L)] = zero16
        return 0
    lax.fori_loop(0, nrows, fill, 0)


def _zero_shared(tmp, shared, r0, chunk_rows):
    off = 0
    while off < RPT:
        n = min(chunk_rows, RPT - off)
        pltpu.sync_copy(tmp.at[pl.ds(0, n)], shared.at[pl.ds(r0 + off, n)])
        off += n


def _sc_edge_body(x_hbm, pk_hbm, w_hbm, s_out,
                  pka, wv0, wv1, rowi0, rowi1, coli0, coli1, rows0, rows1,
                  acc_sh, sem0, sem1, wsem0, wsem1):
    cid = lax.axis_index("c")
    sid = lax.axis_index("s")
    wid = cid * NS + sid
    r0 = sid * RPT

    _zero_fill(rows0, C2)
    _zero_shared(rows0, acc_sh, r0, C2)

    pltpu.sync_copy(pk_hbm.at[wid], pka)
    plsc.subcore_barrier()

    mask = jnp.int32((1 << IDX_BITS) - 1)

    def unpack_row(j, dst):
        for g in range(C2 // L):
            sl = pl.ds(g * L, L)
            dst[sl] = pka[j, sl] & mask

    def gather(ridx, dst, sem):
        return pltpu.make_async_copy(x_hbm.at[ridx], dst, sem)

    def wfetch(j, wvb, wsem):
        base = (wid * NCH2 + j) * C2
        return pltpu.make_async_copy(w_hbm.at[pl.ds(base, C2)], wvb, wsem)

    def process(j, buf, wvb, wsem, colib):
        wfetch(j, wvb, wsem).wait()
        for g in range(C2 // L):
            sl = pl.ds(g * L, L)
            colib[sl] = pka[j, sl] >> IDX_BITS
        def scale(g, _):
            wg = wvb[pl.ds(g * L, L)]
            for lane in range(L):
                i = g * L + lane
                wb = jnp.full((L,), wg[lane], jnp.float32)
                for f in range(D // L):
                    sl = pl.ds(f * L, L)
                    buf[i, sl] = buf[i, sl] * wb
            return 0
        lax.fori_loop(0, C2 // L, scale, 0)

    wfetch(0, wv0, wsem0).start()
    wfetch(1, wv1, wsem1).start()
    unpack_row(0, rowi0)
    gather(rowi0, rows0, sem0).start()
    unpack_row(1, rowi1)
    gather(rowi1, rows1, sem1).start()

    def pair(i, _):
        j = 2 * i
        gather(rowi0, rows0, sem0).wait()
        process(j, rows0, wv0, wsem0, coli0)
        pltpu.sync_copy(rows0, acc_sh.at[coli0], add=True)
        wfetch(j + 2, wv0, wsem0).start()

        gather(rowi1, rows1, sem1).wait()
        process(j + 1, rows1, wv1, wsem1, coli1)
        pltpu.sync_copy(rows1, acc_sh.at[coli1], add=True)

        unpack_row(j + 2, rowi0)
        gather(rowi0, rows0, sem0).start()

        @pl.when(i < NCH2 // 2 - 1)
        def _():
            wfetch(j + 3, wv1, wsem1).start()
            unpack_row(j + 3, rowi1)
            gather(rowi1, rows1, sem1).start()
        return 0
    lax.fori_loop(0, NCH2 // 2, pair, 0)

    gather(rowi0, rows0, sem0).wait()
    process(NCH2 - 1, rows0, wv0, wsem0, coli0)
    pltpu.sync_copy(rows0, acc_sh.at[coli0], add=True)

    plsc.subcore_barrier()
    pltpu.sync_copy(acc_sh.at[pl.ds(r0, RPT)], s_out.at[cid, pl.ds(r0, RPT)])


_sc_edge = pl.kernel(
    _sc_edge_body,
    out_type=[jax.ShapeDtypeStruct((NC, N_PAD, D), jnp.float32)],
    mesh=plsc.VectorSubcoreMesh(**_MESH),
    scratch_types=[
        pltpu.VMEM((NCH2, C2), jnp.int32),
        pltpu.VMEM((C2,), jnp.float32),
        pltpu.VMEM((C2,), jnp.float32),
        pltpu.VMEM((C2,), jnp.int32),
        pltpu.VMEM((C2,), jnp.int32),
        pltpu.VMEM((C2,), jnp.int32),
        pltpu.VMEM((C2,), jnp.int32),
        pltpu.VMEM((C2, D), jnp.float32),
        pltpu.VMEM((C2, D), jnp.float32),
        pltpu.VMEM_SHARED((N_PAD, D), jnp.float32),
        pltpu.SemaphoreType.DMA,
        pltpu.SemaphoreType.DMA,
        pltpu.SemaphoreType.DMA,
        pltpu.SemaphoreType.DMA,
    ],
)


CW = 128


def _sc_cnt_body(col_hbm, c_out, coli0, coli1, ones, acc_sh,
                 isem0, isem1):
    cid = lax.axis_index("c")
    sid = lax.axis_index("s")
    wid = cid * NS + sid
    r0 = sid * RPT

    zero16 = jnp.zeros((L,), jnp.float32)
    one16 = jnp.ones((L,), jnp.float32)

    def fill(r, _):
        for f in range(CW // L):
            ones[r, pl.ds(f * L, L)] = zero16
        return 0
    lax.fori_loop(0, C, fill, 0)
    off = 0
    while off < RPT:
        n = min(C, RPT - off)
        pltpu.sync_copy(ones.at[pl.ds(0, n)], acc_sh.at[pl.ds(r0 + off, n)])
        off += n

    def fill1(r, _):
        for f in range(CW // L):
            ones[r, pl.ds(f * L, L)] = one16
        return 0
    lax.fori_loop(0, C, fill1, 0)
    plsc.subcore_barrier()

    def ifetch(k, colib, isem):
        base = wid * EPT + k * C
        return pltpu.make_async_copy(col_hbm.at[pl.ds(base, C)], colib, isem)

    ifetch(0, coli0, isem0).start()
    ifetch(1, coli1, isem1).start()

    def pair(i, _):
        j = 2 * i
        ifetch(j, coli0, isem0).wait()
        pltpu.sync_copy(ones, acc_sh.at[coli0], add=True)
        ifetch(j + 2, coli0, isem0).start()
        ifetch(j + 1, coli1, isem1).wait()
        pltpu.sync_copy(ones, acc_sh.at[coli1], add=True)

        @pl.when(i < NCH // 2 - 1)
        def _():
            ifetch(j + 3, coli1, isem1).start()
        return 0
    lax.fori_loop(0, NCH // 2, pair, 0)

    ifetch(NCH - 1, coli0, isem0).wait()
    pltpu.sync_copy(ones, acc_sh.at[coli0], add=True)

    plsc.subcore_barrier()
    pltpu.sync_copy(acc_sh.at[pl.ds(r0, RPT)], c_out.at[cid, pl.ds(r0, RPT)])


_sc_cnt = pl.kernel(
    _sc_cnt_body,
    out_type=[jax.ShapeDtypeStruct((NC, N_PAD, CW), jnp.float32)],
    mesh=plsc.VectorSubcoreMesh(**_MESH),
    scratch_types=[
        pltpu.VMEM((C,), jnp.int32),
        pltpu.VMEM((C,), jnp.int32),
        pltpu.VMEM((C, CW), jnp.float32),
        pltpu.VMEM_SHARED((N_PAD, CW), jnp.float32),
        pltpu.SemaphoreType.DMA,
        pltpu.SemaphoreType.DMA,
    ],
)


_EROWS = E // D


def _prep_body(row_ref, col_ref, ew_ref, pk_ref, w_ref):
    r = row_ref[...]
    c = col_ref[...]
    pk_ref[...] = r | (c << IDX_BITS)
    a = jnp.where(r > CELL_LEN, jnp.float32(1), jnp.float32(0))
    b = jnp.where(c > CELL_LEN, jnp.float32(1), jnp.float32(0))
    cross = jnp.abs(a - b)
    tw = jnp.float32(SAME_W) + jnp.float32(CROSS_W - SAME_W) * cross
    w_ref[...] = tw * ew_ref[...]


def _tc_prep(row, col, ew):
    pk, w = pl.pallas_call(
        _prep_body,
        out_shape=[jax.ShapeDtypeStruct((_EROWS, D), jnp.int32),
                   jax.ShapeDtypeStruct((_EROWS, D), jnp.float32)],
    )(row.reshape(_EROWS, D), col.reshape(_EROWS, D), ew.reshape(_EROWS, D))
    return pk.reshape(NW, NCH2, C2), w.reshape(E)


_RB = 1000


def _combine_body(s_ref, cnt_ref, x_ref, wr_ref, out_ref):
    c = cnt_ref[0, :, 0:1] + cnt_ref[1, :, 0:1]
    inv = 1.0 / jnp.maximum(c, 1.0)
    agg = (s_ref[0] + s_ref[1]) * inv
    root = lax.dot_general(x_ref[...], wr_ref[...],
                           (((1,), (1,)), ((), ())),
                           preferred_element_type=jnp.float32)
    out_ref[...] = agg + root


def _tc_combine(s, cnt, x, Wr):
    return pl.pallas_call(
        _combine_body,
        grid=(N // _RB,),
        in_specs=[
            pl.BlockSpec((NC, _RB, D), lambda i: (0, i, 0)),
            pl.BlockSpec((NC, _RB, CW), lambda i: (0, i, 0)),
            pl.BlockSpec((_RB, D), lambda i: (i, 0)),
            pl.BlockSpec((D, D), lambda i: (0, 0)),
        ],
        out_specs=pl.BlockSpec((_RB, D), lambda i: (i, 0)),
        out_shape=jax.ShapeDtypeStruct((N, D), jnp.float32),
    )(s, cnt, x, Wr)


def _first(r):
    return r[0] if isinstance(r, (tuple, list)) else r


def kernel(x, edge_index, edge_weight, Wr1, Wr2):
    row = edge_index[0]
    col = edge_index[1]
    pk3, w3 = _tc_prep(row, col, edge_weight)
    cnt = _first(_sc_cnt(col))
    s1 = _first(_sc_edge(x, pk3, w3))
    h = _tc_combine(s1, cnt, x, Wr1)
    s2 = _first(_sc_edge(h, pk3, w3))
    out = _tc_combine(s2, cnt, h, Wr2)
    return out


# speedup vs baseline: 1.3274x; 1.3274x over previous
import jax
import jax.numpy as jnp
from jax import lax
from jax.experimental import pallas as pl
from jax.experimental.pallas import tpu as pltpu
from jax.experimental.pallas import tpu_sc as plsc

N = 10000
D = 128
E = 320000
CELL_LEN = 16
SAME_W = 0.3
CROSS_W = 1.0

NC = 2
NS = 16
NW = NC * NS
L = 16

EPT = E // NW
C = 80
NCH = EPT // C
N_PAD = 10240
RPT = N_PAD // NS

C2 = 80
NCH2 = 125
IDX_BITS = 14

_MESH = dict(core_axis_name="c", subcore_axis_name="s",
             num_cores=NC, num_subcores=NS)


def _zero_fill(buf, nrows):
    zero16 = jnp.zeros((L,), jnp.float32)

    def fill(r, _):
        for f in range(D // L):
            buf[r, pl.ds(f * L, L)] = zero16
        return 0
    lax.fori_loop(0, nrows, fill, 0)


def _zero_shared(tmp, shared, r0, chunk_rows):
    off = 0
    while off < RPT:
        n = min(chunk_rows, RPT - off)
        pltpu.sync_copy(tmp.at[pl.ds(0, n)], shared.at[pl.ds(r0 + off, n)])
        off += n


def _sc_edge_body(x_hbm, pk_hbm, w_hbm, s_out,
                  pka, wv0, wv1, rowi0, rowi1, coli0, coli1, rows0, rows1,
                  acc_sh, sem0, sem1, wsem0, wsem1):
    cid = lax.axis_index("c")
    sid = lax.axis_index("s")
    wid = cid * NS + sid
    r0 = sid * RPT

    _zero_fill(rows0, C2)
    _zero_shared(rows0, acc_sh, r0, C2)

    pltpu.sync_copy(pk_hbm.at[wid], pka)
    plsc.subcore_barrier()

    mask = jnp.int32((1 << IDX_BITS) - 1)

    def unpack_row(j, dst):
        for g in range(C2 // L):
            sl = pl.ds(g * L, L)
            dst[sl] = pka[j, sl] & mask

    def gather(ridx, dst, sem):
        return pltpu.make_async_copy(x_hbm.at[ridx], dst, sem)

    def wfetch(j, wvb, wsem):
        base = (wid * NCH2 + j) * C2
        return pltpu.make_async_copy(w_hbm.at[pl.ds(base, C2)], wvb, wsem)

    def process(j, buf, wvb, wsem, colib):
        wfetch(j, wvb, wsem).wait()
        for g in range(C2 // L):
            sl = pl.ds(g * L, L)
            colib[sl] = pka[j, sl] >> IDX_BITS
        def scale(g, _):
            wg = wvb[pl.ds(g * L, L)]
            for lane in range(L):
                i = g * L + lane
                wb = jnp.full((L,), wg[lane], jnp.float32)
                for f in range(D // L):
                    sl = pl.ds(f * L, L)
                    buf[i, sl] = buf[i, sl] * wb
            return 0
        lax.fori_loop(0, C2 // L, scale, 0)

    wfetch(0, wv0, wsem0).start()
    wfetch(1, wv1, wsem1).start()
    unpack_row(0, rowi0)
    gather(rowi0, rows0, sem0).start()
    unpack_row(1, rowi1)
    gather(rowi1, rows1, sem1).start()

    def pair(i, _):
        j = 2 * i
        gather(rowi0, rows0, sem0).wait()
        process(j, rows0, wv0, wsem0, coli0)
        pltpu.sync_copy(rows0, acc_sh.at[coli0], add=True)
        wfetch(j + 2, wv0, wsem0).start()
        unpack_row(j + 2, rowi0)
        gather(rowi0, rows0, sem0).start()

        gather(rowi1, rows1, sem1).wait()
        process(j + 1, rows1, wv1, wsem1, coli1)
        pltpu.sync_copy(rows1, acc_sh.at[coli1], add=True)

        @pl.when(i < NCH2 // 2 - 1)
        def _():
            wfetch(j + 3, wv1, wsem1).start()
            unpack_row(j + 3, rowi1)
            gather(rowi1, rows1, sem1).start()
        return 0
    lax.fori_loop(0, NCH2 // 2, pair, 0)

    gather(rowi0, rows0, sem0).wait()
    process(NCH2 - 1, rows0, wv0, wsem0, coli0)
    pltpu.sync_copy(rows0, acc_sh.at[coli0], add=True)

    plsc.subcore_barrier()
    pltpu.sync_copy(acc_sh.at[pl.ds(r0, RPT)], s_out.at[cid, pl.ds(r0, RPT)])


_sc_edge = pl.kernel(
    _sc_edge_body,
    out_type=[jax.ShapeDtypeStruct((NC, N_PAD, D), jnp.float32)],
    mesh=plsc.VectorSubcoreMesh(**_MESH),
    scratch_types=[
        pltpu.VMEM((NCH2, C2), jnp.int32),
        pltpu.VMEM((C2,), jnp.float32),
        pltpu.VMEM((C2,), jnp.float32),
        pltpu.VMEM((C2,), jnp.int32),
        pltpu.VMEM((C2,), jnp.int32),
        pltpu.VMEM((C2,), jnp.int32),
        pltpu.VMEM((C2,), jnp.int32),
        pltpu.VMEM((C2, D), jnp.float32),
        pltpu.VMEM((C2, D), jnp.float32),
        pltpu.VMEM_SHARED((N_PAD, D), jnp.float32),
        pltpu.SemaphoreType.DMA,
        pltpu.SemaphoreType.DMA,
        pltpu.SemaphoreType.DMA,
        pltpu.SemaphoreType.DMA,
    ],
)


CW = 128


def _sc_cnt_body(col_hbm, c_out, coli0, coli1, ones, acc_sh,
                 isem0, isem1):
    cid = lax.axis_index("c")
    sid = lax.axis_index("s")
    wid = cid * NS + sid
    r0 = sid * RPT

    zero16 = jnp.zeros((L,), jnp.float32)
    one16 = jnp.ones((L,), jnp.float32)

    def fill(r, _):
        for f in range(CW // L):
            ones[r, pl.ds(f * L, L)] = zero16
        return 0
    lax.fori_loop(0, C, fill, 0)
    off = 0
    while off < RPT:
        n = min(C, RPT - off)
        pltpu.sync_copy(ones.at[pl.ds(0, n)], acc_sh.at[pl.ds(r0 + off, n)])
        off += n

    def fill1(r, _):
        for f in range(CW // L):
            ones[r, pl.ds(f * L, L)] = one16
        return 0
    lax.fori_loop(0, C, fill1, 0)
    plsc.subcore_barrier()

    def ifetch(k, colib, isem):
        base = wid * EPT + k * C
        return pltpu.make_async_copy(col_hbm.at[pl.ds(base, C)], colib, isem)

    ifetch(0, coli0, isem0).start()
    ifetch(1, coli1, isem1).start()

    def pair(i, _):
        j = 2 * i
        ifetch(j, coli0, isem0).wait()
        pltpu.sync_copy(ones, acc_sh.at[coli0], add=True)
        ifetch(j + 2, coli0, isem0).start()
        ifetch(j + 1, coli1, isem1).wait()
        pltpu.sync_copy(ones, acc_sh.at[coli1], add=True)

        @pl.when(i < NCH // 2 - 1)
        def _():
            ifetch(j + 3, coli1, isem1).start()
        return 0
    lax.fori_loop(0, NCH // 2, pair, 0)

    ifetch(NCH - 1, coli0, isem0).wait()
    pltpu.sync_copy(ones, acc_sh.at[coli0], add=True)

    plsc.subcore_barrier()
    pltpu.sync_copy(acc_sh.at[pl.ds(r0, RPT)], c_out.at[cid, pl.ds(r0, RPT)])


_sc_cnt = pl.kernel(
    _sc_cnt_body,
    out_type=[jax.ShapeDtypeStruct((NC, N_PAD, CW), jnp.float32)],
    mesh=plsc.VectorSubcoreMesh(**_MESH),
    scratch_types=[
        pltpu.VMEM((C,), jnp.int32),
        pltpu.VMEM((C,), jnp.int32),
        pltpu.VMEM((C, CW), jnp.float32),
        pltpu.VMEM_SHARED((N_PAD, CW), jnp.float32),
        pltpu.SemaphoreType.DMA,
        pltpu.SemaphoreType.DMA,
    ],
)


_EROWS = E // D


def _prep_body(row_ref, col_ref, ew_ref, pk_ref, w_ref):
    r = row_ref[...]
    c = col_ref[...]
    pk_ref[...] = r | (c << IDX_BITS)
    a = jnp.where(r > CELL_LEN, jnp.float32(1), jnp.float32(0))
    b = jnp.where(c > CELL_LEN, jnp.float32(1), jnp.float32(0))
    cross = jnp.abs(a - b)
    tw = jnp.float32(SAME_W) + jnp.float32(CROSS_W - SAME_W) * cross
    w_ref[...] = tw * ew_ref[...]


def _tc_prep(row, col, ew):
    pk, w = pl.pallas_call(
        _prep_body,
        out_shape=[jax.ShapeDtypeStruct((_EROWS, D), jnp.int32),
                   jax.ShapeDtypeStruct((_EROWS, D), jnp.float32)],
    )(row.reshape(_EROWS, D), col.reshape(_EROWS, D), ew.reshape(_EROWS, D))
    return pk.reshape(NW, NCH2, C2), w.reshape(E)


_RB = 1000


def _combine_body(s_ref, cnt_ref, x_ref, wr_ref, out_ref):
    c = cnt_ref[0, :, 0:1] + cnt_ref[1, :, 0:1]
    inv = 1.0 / jnp.maximum(c, 1.0)
    agg = (s_ref[0] + s_ref[1]) * inv
    root = lax.dot_general(x_ref[...], wr_ref[...],
                           (((1,), (1,)), ((), ())),
                           preferred_element_type=jnp.float32)
    out_ref[...] = agg + root


def _tc_combine(s, cnt, x, Wr):
    return pl.pallas_call(
        _combine_body,
        grid=(N // _RB,),
        in_specs=[
            pl.BlockSpec((NC, _RB, D), lambda i: (0, i, 0)),
            pl.BlockSpec((NC, _RB, CW), lambda i: (0, i, 0)),
            pl.BlockSpec((_RB, D), lambda i: (i, 0)),
            pl.BlockSpec((D, D), lambda i: (0, 0)),
        ],
        out_specs=pl.BlockSpec((_RB, D), lambda i: (i, 0)),
        out_shape=jax.ShapeDtypeStruct((N, D), jnp.float32),
    )(s, cnt, x, Wr)


def _first(r):
    return r[0] if isinstance(r, (tuple, list)) else r


def kernel(x, edge_index, edge_weight, Wr1, Wr2):
    row = edge_index[0]
    col = edge_index[1]
    pk3, w3 = _tc_prep(row, col, edge_weight)
    cnt = _first(_sc_cnt(col))
    s1 = _first(_sc_edge(x, pk3, w3))
    h = _tc_combine(s1, cnt, x, Wr1)
    s2 = _first(_sc_edge(h, pk3, w3))
    out = _tc_combine(s2, cnt, h, Wr2)
    return out


# speedup vs baseline: 1.3434x; 1.0121x over previous
import jax
import jax.numpy as jnp
from jax import lax
from jax.experimental import pallas as pl
from jax.experimental.pallas import tpu as pltpu
from jax.experimental.pallas import tpu_sc as plsc

N = 10000
D = 128
E = 320000
CELL_LEN = 16
SAME_W = 0.3
CROSS_W = 1.0

NC = 2
NS = 16
NW = NC * NS
L = 16

EPT = E // NW
C = 80
NCH = EPT // C
N_PAD = 10240
RPT = N_PAD // NS

C2 = 80
CA = 48
NCH2 = 125
IDX_BITS = 14

_MESH = dict(core_axis_name="c", subcore_axis_name="s",
             num_cores=NC, num_subcores=NS)


def _zero_fill(buf, nrows):
    zero16 = jnp.zeros((L,), jnp.float32)

    def fill(r, _):
        for f in range(D // L):
            buf[r, pl.ds(f * L, L)] = zero16
        return 0
    lax.fori_loop(0, nrows, fill, 0)


def _zero_shared(tmp, shared, r0, chunk_rows):
    off = 0
    while off < RPT:
        n = min(chunk_rows, RPT - off)
        pltpu.sync_copy(tmp.at[pl.ds(0, n)], shared.at[pl.ds(r0 + off, n)])
        off += n


def _sc_edge_body(x_hbm, pk_hbm, w_hbm, s_out,
                  pka, wv0, wv1, rowi0, rowi1, coliA, coliB, rows0, rows1,
                  acc_sh, sem0, sem1, wsem0, wsem1, ssem):
    cid = lax.axis_index("c")
    sid = lax.axis_index("s")
    wid = cid * NS + sid
    r0 = sid * RPT

    _zero_fill(rows0, C2)
    _zero_shared(rows0, acc_sh, r0, C2)

    pltpu.sync_copy(pk_hbm.at[wid], pka)
    plsc.subcore_barrier()

    mask = jnp.int32((1 << IDX_BITS) - 1)

    def unpack_row(j, dst):
        for g in range(C2 // L):
            sl = pl.ds(g * L, L)
            dst[sl] = pka[j, sl] & mask

    def gather(ridx, dst, sem):
        return pltpu.make_async_copy(x_hbm.at[ridx], dst, sem)

    def wfetch(j, wvb, wsem):
        base = (wid * NCH2 + j) * C2
        return pltpu.make_async_copy(w_hbm.at[pl.ds(base, C2)], wvb, wsem)

    def process(j, buf, wvb, wsem, coliA, coliB, ssem):
        wfetch(j, wvb, wsem).wait()
        for g in range(C2 // L):
            sl = pl.ds(g * L, L)
            if g * L < CA:
                coliA[sl] = pka[j, sl] >> IDX_BITS
            else:
                coliB[pl.ds(g * L - CA, L)] = pka[j, sl] >> IDX_BITS
        def scale(g, _):
            wg = wvb[pl.ds(g * L, L)]
            for lane in range(L):
                i = g * L + lane
                wb = jnp.full((L,), wg[lane], jnp.float32)
                for f in range(D // L):
                    sl = pl.ds(f * L, L)
                    buf[i, sl] = buf[i, sl] * wb
            return 0
        lax.fori_loop(0, CA // L, scale, 0)
        hA = pltpu.async_copy(buf.at[pl.ds(0, CA)], acc_sh.at[coliA],
                              ssem, add=True)
        lax.fori_loop(CA // L, C2 // L, scale, 0)
        hA.wait()
        pltpu.sync_copy(buf.at[pl.ds(CA, C2 - CA)], acc_sh.at[coliB],
                        add=True)

    wfetch(0, wv0, wsem0).start()
    wfetch(1, wv1, wsem1).start()
    unpack_row(0, rowi0)
    gather(rowi0, rows0, sem0).start()
    unpack_row(1, rowi1)
    gather(rowi1, rows1, sem1).start()

    def pair(i, _):
        j = 2 * i
        gather(rowi0, rows0, sem0).wait()
        process(j, rows0, wv0, wsem0, coliA, coliB, ssem)
        wfetch(j + 2, wv0, wsem0).start()
        unpack_row(j + 2, rowi0)
        gather(rowi0, rows0, sem0).start()

        gather(rowi1, rows1, sem1).wait()
        process(j + 1, rows1, wv1, wsem1, coliA, coliB, ssem)

        @pl.when(i < NCH2 // 2 - 1)
        def _():
            wfetch(j + 3, wv1, wsem1).start()
            unpack_row(j + 3, rowi1)
            gather(rowi1, rows1, sem1).start()
        return 0
    lax.fori_loop(0, NCH2 // 2, pair, 0)

    gather(rowi0, rows0, sem0).wait()
    process(NCH2 - 1, rows0, wv0, wsem0, coliA, coliB, ssem)

    plsc.subcore_barrier()
    pltpu.sync_copy(acc_sh.at[pl.ds(r0, RPT)], s_out.at[cid, pl.ds(r0, RPT)])


_sc_edge = pl.kernel(
    _sc_edge_body,
    out_type=[jax.ShapeDtypeStruct((NC, N_PAD, D), jnp.float32)],
    mesh=plsc.VectorSubcoreMesh(**_MESH),
    scratch_types=[
        pltpu.VMEM((NCH2, C2), jnp.int32),
        pltpu.VMEM((C2,), jnp.float32),
        pltpu.VMEM((C2,), jnp.float32),
        pltpu.VMEM((C2,), jnp.int32),
        pltpu.VMEM((C2,), jnp.int32),
        pltpu.VMEM((CA,), jnp.int32),
        pltpu.VMEM((C2 - CA,), jnp.int32),
        pltpu.VMEM((C2, D), jnp.float32),
        pltpu.VMEM((C2, D), jnp.float32),
        pltpu.VMEM_SHARED((N_PAD, D), jnp.float32),
        pltpu.SemaphoreType.DMA,
        pltpu.SemaphoreType.DMA,
        pltpu.SemaphoreType.DMA,
        pltpu.SemaphoreType.DMA,
        pltpu.SemaphoreType.DMA,
    ],
)


CW = 128


def _sc_cnt_body(col_hbm, c_out, coli0, coli1, ones, acc_sh,
                 isem0, isem1):
    cid = lax.axis_index("c")
    sid = lax.axis_index("s")
    wid = cid * NS + sid
    r0 = sid * RPT

    zero16 = jnp.zeros((L,), jnp.float32)
    one16 = jnp.ones((L,), jnp.float32)

    def fill(r, _):
        for f in range(CW // L):
            ones[r, pl.ds(f * L, L)] = zero16
        return 0
    lax.fori_loop(0, C, fill, 0)
    off = 0
    while off < RPT:
        n = min(C, RPT - off)
        pltpu.sync_copy(ones.at[pl.ds(0, n)], acc_sh.at[pl.ds(r0 + off, n)])
        off += n

    def fill1(r, _):
        for f in range(CW // L):
            ones[r, pl.ds(f * L, L)] = one16
        return 0
    lax.fori_loop(0, C, fill1, 0)
    plsc.subcore_barrier()

    def ifetch(k, colib, isem):
        base = wid * EPT + k * C
        return pltpu.make_async_copy(col_hbm.at[pl.ds(base, C)], colib, isem)

    ifetch(0, coli0, isem0).start()
    ifetch(1, coli1, isem1).start()

    def pair(i, _):
        j = 2 * i
        ifetch(j, coli0, isem0).wait()
        pltpu.sync_copy(ones, acc_sh.at[coli0], add=True)
        ifetch(j + 2, coli0, isem0).start()
        ifetch(j + 1, coli1, isem1).wait()
        pltpu.sync_copy(ones, acc_sh.at[coli1], add=True)

        @pl.when(i < NCH // 2 - 1)
        def _():
            ifetch(j + 3, coli1, isem1).start()
        return 0
    lax.fori_loop(0, NCH // 2, pair, 0)

    ifetch(NCH - 1, coli0, isem0).wait()
    pltpu.sync_copy(ones, acc_sh.at[coli0], add=True)

    plsc.subcore_barrier()
    pltpu.sync_copy(acc_sh.at[pl.ds(r0, RPT)], c_out.at[cid, pl.ds(r0, RPT)])


_sc_cnt = pl.kernel(
    _sc_cnt_body,
    out_type=[jax.ShapeDtypeStruct((NC, N_PAD, CW), jnp.float32)],
    mesh=plsc.VectorSubcoreMesh(**_MESH),
    scratch_types=[
        pltpu.VMEM((C,), jnp.int32),
        pltpu.VMEM((C,), jnp.int32),
        pltpu.VMEM((C, CW), jnp.float32),
        pltpu.VMEM_SHARED((N_PAD, CW), jnp.float32),
        pltpu.SemaphoreType.DMA,
        pltpu.SemaphoreType.DMA,
    ],
)


_EROWS = E // D


def _prep_body(row_ref, col_ref, ew_ref, pk_ref, w_ref):
    r = row_ref[...]
    c = col_ref[...]
    pk_ref[...] = r | (c << IDX_BITS)
    a = jnp.where(r > CELL_LEN, jnp.float32(1), jnp.float32(0))
    b = jnp.where(c > CELL_LEN, jnp.float32(1), jnp.float32(0))
    cross = jnp.abs(a - b)
    tw = jnp.float32(SAME_W) + jnp.float32(CROSS_W - SAME_W) * cross
    w_ref[...] = tw * ew_ref[...]


def _tc_prep(row, col, ew):
    pk, w = pl.pallas_call(
        _prep_body,
        out_shape=[jax.ShapeDtypeStruct((_EROWS, D), jnp.int32),
                   jax.ShapeDtypeStruct((_EROWS, D), jnp.float32)],
    )(row.reshape(_EROWS, D), col.reshape(_EROWS, D), ew.reshape(_EROWS, D))
    return pk.reshape(NW, NCH2, C2), w.reshape(E)


_RB = 1000


def _combine_body(s_ref, cnt_ref, x_ref, wr_ref, out_ref):
    c = cnt_ref[0, :, 0:1] + cnt_ref[1, :, 0:1]
    inv = 1.0 / jnp.maximum(c, 1.0)
    agg = (s_ref[0] + s_ref[1]) * inv
    root = lax.dot_general(x_ref[...], wr_ref[...],
                           (((1,), (1,)), ((), ())),
                           preferred_element_type=jnp.float32)
    out_ref[...] = agg + root


def _tc_combine(s, cnt, x, Wr):
    return pl.pallas_call(
        _combine_body,
        grid=(N // _RB,),
        in_specs=[
            pl.BlockSpec((NC, _RB, D), lambda i: (0, i, 0)),
            pl.BlockSpec((NC, _RB, CW), lambda i: (0, i, 0)),
            pl.BlockSpec((_RB, D), lambda i: (i, 0)),
            pl.BlockSpec((D, D), lambda i: (0, 0)),
        ],
        out_specs=pl.BlockSpec((_RB, D), lambda i: (i, 0)),
        out_shape=jax.ShapeDtypeStruct((N, D), jnp.float32),
    )(s, cnt, x, Wr)


def _first(r):
    return r[0] if isinstance(r, (tuple, list)) else r


def kernel(x, edge_index, edge_weight, Wr1, Wr2):
    row = edge_index[0]
    col = edge_index[1]
    pk3, w3 = _tc_prep(row, col, edge_weight)
    cnt = _first(_sc_cnt(col))
    s1 = _first(_sc_edge(x, pk3, w3))
    h = _tc_combine(s1, cnt, x, Wr1)
    s2 = _first(_sc_edge(h, pk3, w3))
    out = _tc_combine(s2, cnt, h, Wr2)
    return out
